# baseline (speedup 1.0000x reference)
import functools

import jax
import jax.numpy as jnp
from jax import lax
from jax.experimental import pallas as pl
from jax.experimental.pallas import tpu as pltpu
from jax.experimental.pallas import tpu_sc as plsc

N = 10000
E = 320000
D = 128
H = 64

NC = 2
NS = 16
NW = NC * NS
RPT = 632
NPAD = NS * RPT

KD = 256
DCH = 40
KS = 128
SCH = 160
HCH = SCH // 2

_mesh = plsc.VectorSubcoreMesh(core_axis_name="c", subcore_axis_name="s")


@functools.partial(
    pl.kernel,
    out_type=jax.ShapeDtypeStruct((NC * NPAD, 16), jnp.float32),
    mesh=_mesh,
    scratch_types=[
        pltpu.VMEM((DCH, KD), jnp.int32),
        pltpu.VMEM((KD, 16), jnp.float32),
        pltpu.VMEM_SHARED((NPAD, 16), jnp.float32),
    ],
    compiler_params=pltpu.CompilerParams(use_tc_tiling_on_sc=False),
)
def _sc_degree(d_hbm, zeros16_hbm, ones16_hbm, out_hbm, d_v, ones_v, acc):
    c = lax.axis_index("c")
    t = lax.axis_index("s")
    w = c * NS + t
    pltpu.sync_copy(d_hbm.at[pl.ds(w * DCH, DCH)], d_v)
    pltpu.sync_copy(ones16_hbm, ones_v)
    pltpu.sync_copy(zeros16_hbm, acc.at[pl.ds(t * RPT, RPT)])
    plsc.subcore_barrier()

    def chunk(i, carry):
        pltpu.sync_copy(ones_v, acc.at[d_v.at[i]], add=True)
        return carry

    lax.fori_loop(0, DCH, chunk, 0)
    plsc.subcore_barrier()
    pltpu.sync_copy(acc.at[pl.ds(t * RPT, RPT)],
                    out_hbm.at[pl.ds(c * NPAD + t * RPT, RPT)])


@functools.partial(
    pl.kernel,
    out_type=jax.ShapeDtypeStruct((NC * NPAD, H), jnp.float32),
    mesh=_mesh,
    scratch_types=[
        pltpu.VMEM((HCH, KS), jnp.int32),
        pltpu.VMEM((HCH, KS), jnp.int32),
        pltpu.VMEM((KS, H), jnp.float32),
        pltpu.VMEM((KS, H), jnp.float32),
        pltpu.VMEM_SHARED((NPAD, H), jnp.float32),
        pltpu.VMEM_SHARED((NPAD, H), jnp.float32),
        pltpu.SemaphoreType.DMA,
        pltpu.SemaphoreType.DMA,
        pltpu.SemaphoreType.DMA,
        pltpu.SemaphoreType.DMA,
    ],
    compiler_params=pltpu.CompilerParams(use_tc_tiling_on_sc=False),
)
def _sc_scatter(x_hbm, s_hbm, d_hbm, zeros_hbm, out_hbm,
                s_v, d_v, rows0, rows1, xs, acc, sem0, sem1, sems0, sems1):
    c = lax.axis_index("c")
    t = lax.axis_index("s")
    pltpu.sync_copy(x_hbm.at[pl.ds(c * NPAD + t * RPT, RPT)],
                    xs.at[pl.ds(t * RPT, RPT)])
    pltpu.sync_copy(zeros_hbm, acc.at[pl.ds(t * RPT, RPT)])
    plsc.subcore_barrier()

    for half in range(2):
        base = t * SCH + half * HCH
        pltpu.sync_copy(s_hbm.at[pl.ds(base, HCH)], s_v)
        pltpu.sync_copy(d_hbm.at[pl.ds(base, HCH)], d_v)

        pltpu.async_copy(xs.at[s_v.at[0]], rows0, sem0)

        def body(i, carry):
            a = 2 * i
            pltpu.async_copy(xs.at[s_v.at[a + 1]], rows1, sem1)
            pltpu.make_async_copy(xs.at[s_v.at[a]], rows0, sem0).wait()
            pltpu.async_copy(rows0, acc.at[d_v.at[a]], sems0, add=True)
            pltpu.make_async_copy(xs.at[s_v.at[a + 1]], rows1, sem1).wait()
            pltpu.async_copy(rows1, acc.at[d_v.at[a + 1]], sems1, add=True)
            pltpu.make_async_copy(rows0, acc.at[d_v.at[a]], sems0).wait()

            @pl.when(a + 2 < HCH)
            def _():
                pltpu.async_copy(xs.at[s_v.at[a + 2]], rows0, sem0)

            pltpu.make_async_copy(rows1, acc.at[d_v.at[a + 1]],
                                  sems1).wait()
            return carry

        lax.fori_loop(0, HCH // 2, body, 0)

    plsc.subcore_barrier()
    pltpu.sync_copy(acc.at[pl.ds(t * RPT, RPT)],
                    out_hbm.at[pl.ds(c * NPAD + t * RPT, RPT)])


def _dinv_from(degp_ref):
    indeg = degp_ref[0:N, 0:1] + degp_ref[NPAD:NPAD + N, 0:1]
    deg = indeg + 1.0
    return lax.rsqrt(jnp.maximum(deg, 1.0))


def _bn_in(h, gamma, beta, eps=1e-3):
    mu = jnp.mean(h, axis=0, keepdims=True)
    var = jnp.mean((h - mu) * (h - mu), axis=0, keepdims=True)
    return (h - mu) * lax.rsqrt(var + eps) * gamma + beta


def _leaky_in(h, alpha=0.3):
    return jnp.where(h > 0, h, alpha * h)


def _join_cols(ref):
    return jnp.concatenate([ref[0:N, :], ref[NPAD:NPAD + N, :]], axis=1)


def _store_split(ref, m):
    ref[0:N, :] = m[:, 0:H]
    ref[NPAD:NPAD + N, :] = m[:, H:D]


def _tc1(x_ref, degp_ref, g1_ref, b1_ref, x1_ref):
    h0 = _bn_in(x_ref[...], g1_ref[...], b1_ref[...])
    _store_split(x1_ref, _dinv_from(degp_ref) * h0)


def _tc2(s1_ref, x1_ref, degp_ref, w1_ref, b1_ref, w2_ref,
         g2_ref, bt2_ref, ws_ref, bs_ref, x2_ref, skip_ref):
    dinv = _dinv_from(degp_ref)
    g0 = dinv * (_join_cols(s1_ref) + _join_cols(x1_ref))
    h1 = _leaky_in(jnp.dot(g0, w1_ref[...],
                           preferred_element_type=jnp.float32) + b1_ref[...])
    h1b = _bn_in(h1, g2_ref[...], bt2_ref[...])
    _store_split(x2_ref, dinv * jnp.dot(h1b, w2_ref[...],
                                        preferred_element_type=jnp.float32))
    skip_ref[...] = jnp.dot(g0, ws_ref[...],
                            preferred_element_type=jnp.float32) + bs_ref[...]


def _tc3(s2_ref, x2_ref, degp_ref, b2_ref, skip_ref, out_ref):
    h = _leaky_in(_dinv_from(degp_ref)
                  * (_join_cols(s2_ref) + _join_cols(x2_ref)) + b2_ref[...])
    out_ref[...] = h + skip_ref[...]


def _tc_call(f, out_shapes):
    return pl.pallas_call(
        f, out_shape=[jax.ShapeDtypeStruct(s, jnp.float32) for s in out_shapes])


@jax.jit
def kernel(x, edge_index, W1, b1, W2, b2, Ws, bs, gamma1, beta1, gamma2,
           beta2):
    ei = edge_index.astype(jnp.int32)
    src, dst = ei[0], ei[1]

    pad_deg = NW * DCH * KD - E
    d_deg = jnp.concatenate([dst, jnp.full((pad_deg,), N, jnp.int32)])
    d_deg = d_deg.reshape(NW * DCH, KD)

    pad_sc = NS * SCH * KS - E
    s_hbm = jnp.concatenate([src, jnp.zeros((pad_sc,), jnp.int32)])
    d_hbm = jnp.concatenate([dst, jnp.full((pad_sc,), N, jnp.int32)])
    s_hbm = s_hbm.reshape(NS * SCH, KS)
    d_hbm = d_hbm.reshape(NS * SCH, KS)

    zeros_rows = jnp.zeros((RPT, H), jnp.float32)
    zeros16 = jnp.zeros((RPT, 16), jnp.float32)
    ones16 = jnp.ones((KD, 16), jnp.float32)

    degp = _sc_degree(d_deg, zeros16, ones16)

    row = lambda v: v.reshape(1, D)
    (x1,) = _tc_call(_tc1, [(NC * NPAD, H)])(x, degp, row(gamma1), row(beta1))
    s1 = _sc_scatter(x1, s_hbm, d_hbm, zeros_rows)
    x2, skip = _tc_call(_tc2, [(NC * NPAD, H), (N, D)])(
        s1, x1, degp, W1, row(b1), W2, row(gamma2), row(beta2), Ws, row(bs))
    s2 = _sc_scatter(x2, s_hbm, d_hbm, zeros_rows)
    (out,) = _tc_call(_tc3, [(N, D)])(s2, x2, degp, row(b2), skip)
    return out


# speedup vs baseline: 1.0132x; 1.0132x over previous
import functools

import jax
import jax.numpy as jnp
from jax import lax
from jax.experimental import pallas as pl
from jax.experimental.pallas import tpu as pltpu
from jax.experimental.pallas import tpu_sc as plsc

N = 10000
E = 320000
D = 128
H = 64

NC = 2
NS = 16
NW = NC * NS
RPT = 632
NPAD = NS * RPT

KD = 256
DCH = 40
KS = 256
SCH = 80
HCH = 16

_mesh = plsc.VectorSubcoreMesh(core_axis_name="c", subcore_axis_name="s")


@functools.partial(
    pl.kernel,
    out_type=jax.ShapeDtypeStruct((NC * NPAD, 16), jnp.float32),
    mesh=_mesh,
    scratch_types=[
        pltpu.VMEM((DCH, KD), jnp.int32),
        pltpu.VMEM((KD, 16), jnp.float32),
        pltpu.VMEM_SHARED((NPAD, 16), jnp.float32),
    ],
    compiler_params=pltpu.CompilerParams(use_tc_tiling_on_sc=False),
)
def _sc_degree(d_hbm, zeros16_hbm, ones16_hbm, out_hbm, d_v, ones_v, acc):
    c = lax.axis_index("c")
    t = lax.axis_index("s")
    w = c * NS + t
    pltpu.sync_copy(d_hbm.at[pl.ds(w * DCH, DCH)], d_v)
    pltpu.sync_copy(ones16_hbm, ones_v)
    pltpu.sync_copy(zeros16_hbm, acc.at[pl.ds(t * RPT, RPT)])
    plsc.subcore_barrier()

    def chunk(i, carry):
        pltpu.sync_copy(ones_v, acc.at[d_v.at[i]], add=True)
        return carry

    lax.fori_loop(0, DCH, chunk, 0)
    plsc.subcore_barrier()
    pltpu.sync_copy(acc.at[pl.ds(t * RPT, RPT)],
                    out_hbm.at[pl.ds(c * NPAD + t * RPT, RPT)])


@functools.partial(
    pl.kernel,
    out_type=jax.ShapeDtypeStruct((NC * NPAD, H), jnp.float32),
    mesh=_mesh,
    scratch_types=[
        pltpu.VMEM((HCH, KS), jnp.int32),
        pltpu.VMEM((HCH, KS), jnp.int32),
        pltpu.VMEM((KS, H), jnp.float32),
        pltpu.VMEM((KS, H), jnp.float32),
        pltpu.VMEM_SHARED((NPAD, H), jnp.float32),
        pltpu.VMEM_SHARED((NPAD, H), jnp.float32),
        pltpu.SemaphoreType.DMA,
        pltpu.SemaphoreType.DMA,
    ],
    compiler_params=pltpu.CompilerParams(use_tc_tiling_on_sc=False),
)
def _sc_scatter(x_hbm, s_hbm, d_hbm, zeros_hbm, out_hbm,
                s_v, d_v, rows0, rows1, xs, acc, sem0, sem1):
    c = lax.axis_index("c")
    t = lax.axis_index("s")
    pltpu.sync_copy(x_hbm.at[pl.ds(c * NPAD + t * RPT, RPT)],
                    xs.at[pl.ds(t * RPT, RPT)])
    pltpu.sync_copy(zeros_hbm, acc.at[pl.ds(t * RPT, RPT)])
    plsc.subcore_barrier()

    for sec in range(SCH // HCH):
        base = t * SCH + sec * HCH
        pltpu.sync_copy(s_hbm.at[pl.ds(base, HCH)], s_v)
        pltpu.sync_copy(d_hbm.at[pl.ds(base, HCH)], d_v)

        pltpu.async_copy(xs.at[s_v.at[0]], rows0, sem0)

        def body(i, carry):
            a = 2 * i
            pltpu.async_copy(xs.at[s_v.at[a + 1]], rows1, sem1)
            pltpu.make_async_copy(xs.at[s_v.at[a]], rows0, sem0).wait()
            pltpu.sync_copy(rows0, acc.at[d_v.at[a]], add=True)

            @pl.when(a + 2 < HCH)
            def _():
                pltpu.async_copy(xs.at[s_v.at[a + 2]], rows0, sem0)

            pltpu.make_async_copy(xs.at[s_v.at[a + 1]], rows1, sem1).wait()
            pltpu.sync_copy(rows1, acc.at[d_v.at[a + 1]], add=True)
            return carry

        lax.fori_loop(0, HCH // 2, body, 0)

    plsc.subcore_barrier()
    pltpu.sync_copy(acc.at[pl.ds(t * RPT, RPT)],
                    out_hbm.at[pl.ds(c * NPAD + t * RPT, RPT)])


def _dinv_from(degp_ref):
    indeg = degp_ref[0:N, 0:1] + degp_ref[NPAD:NPAD + N, 0:1]
    deg = indeg + 1.0
    return lax.rsqrt(jnp.maximum(deg, 1.0))


def _bn_in(h, gamma, beta, eps=1e-3):
    mu = jnp.mean(h, axis=0, keepdims=True)
    var = jnp.mean((h - mu) * (h - mu), axis=0, keepdims=True)
    return (h - mu) * lax.rsqrt(var + eps) * gamma + beta


def _leaky_in(h, alpha=0.3):
    return jnp.where(h > 0, h, alpha * h)


def _join_cols(ref):
    return jnp.concatenate([ref[0:N, :], ref[NPAD:NPAD + N, :]], axis=1)


def _store_split(ref, m):
    ref[0:N, :] = m[:, 0:H]
    ref[NPAD:NPAD + N, :] = m[:, H:D]


def _tc1(x_ref, degp_ref, g1_ref, b1_ref, x1_ref):
    h0 = _bn_in(x_ref[...], g1_ref[...], b1_ref[...])
    _store_split(x1_ref, _dinv_from(degp_ref) * h0)


def _tc2(s1_ref, x1_ref, degp_ref, w1_ref, b1_ref, w2_ref,
         g2_ref, bt2_ref, ws_ref, bs_ref, x2_ref, skip_ref):
    dinv = _dinv_from(degp_ref)
    g0 = dinv * (_join_cols(s1_ref) + _join_cols(x1_ref))
    h1 = _leaky_in(jnp.dot(g0, w1_ref[...],
                           preferred_element_type=jnp.float32) + b1_ref[...])
    h1b = _bn_in(h1, g2_ref[...], bt2_ref[...])
    _store_split(x2_ref, dinv * jnp.dot(h1b, w2_ref[...],
                                        preferred_element_type=jnp.float32))
    skip_ref[...] = jnp.dot(g0, ws_ref[...],
                            preferred_element_type=jnp.float32) + bs_ref[...]


def _tc3(s2_ref, x2_ref, degp_ref, b2_ref, skip_ref, out_ref):
    h = _leaky_in(_dinv_from(degp_ref)
                  * (_join_cols(s2_ref) + _join_cols(x2_ref)) + b2_ref[...])
    out_ref[...] = h + skip_ref[...]


def _tc_call(f, out_shapes):
    return pl.pallas_call(
        f, out_shape=[jax.ShapeDtypeStruct(s, jnp.float32) for s in out_shapes])


@jax.jit
def kernel(x, edge_index, W1, b1, W2, b2, Ws, bs, gamma1, beta1, gamma2,
           beta2):
    ei = edge_index.astype(jnp.int32)
    src, dst = ei[0], ei[1]

    pad_deg = NW * DCH * KD - E
    d_deg = jnp.concatenate([dst, jnp.full((pad_deg,), N, jnp.int32)])
    d_deg = d_deg.reshape(NW * DCH, KD)

    pad_sc = NS * SCH * KS - E
    s_hbm = jnp.concatenate([src, jnp.zeros((pad_sc,), jnp.int32)])
    d_hbm = jnp.concatenate([dst, jnp.full((pad_sc,), N, jnp.int32)])
    s_hbm = s_hbm.reshape(NS * SCH, KS)
    d_hbm = d_hbm.reshape(NS * SCH, KS)

    zeros_rows = jnp.zeros((RPT, H), jnp.float32)
    zeros16 = jnp.zeros((RPT, 16), jnp.float32)
    ones16 = jnp.ones((KD, 16), jnp.float32)

    degp = _sc_degree(d_deg, zeros16, ones16)

    row = lambda v: v.reshape(1, D)
    (x1,) = _tc_call(_tc1, [(NC * NPAD, H)])(x, degp, row(gamma1), row(beta1))
    s1 = _sc_scatter(x1, s_hbm, d_hbm, zeros_rows)
    x2, skip = _tc_call(_tc2, [(NC * NPAD, H), (N, D)])(
        s1, x1, degp, W1, row(b1), W2, row(gamma2), row(beta2), Ws, row(bs))
    s2 = _sc_scatter(x2, s_hbm, d_hbm, zeros_rows)
    (out,) = _tc_call(_tc3, [(N, D)])(s2, x2, degp, row(b2), skip)
    return out


# speedup vs baseline: 1.0410x; 1.0274x over previous
import functools

import jax
import jax.numpy as jnp
from jax import lax
from jax.experimental import pallas as pl
from jax.experimental.pallas import tpu as pltpu
from jax.experimental.pallas import tpu_sc as plsc

N = 10000
E = 320000
D = 128
H = 64

NC = 2
NS = 16
NW = NC * NS
RPT = 632
NPAD = NS * RPT

KD = 256
DCH = 40
KS = 128
SCH = 160
HCH = 80

_mesh = plsc.VectorSubcoreMesh(core_axis_name="c", subcore_axis_name="s")


@functools.partial(
    pl.kernel,
    out_type=jax.ShapeDtypeStruct((NC * NPAD, 16), jnp.float32),
    mesh=_mesh,
    scratch_types=[
        pltpu.VMEM((DCH, KD), jnp.int32),
        pltpu.VMEM((KD, 16), jnp.float32),
        pltpu.VMEM_SHARED((NPAD, 16), jnp.float32),
    ],
    compiler_params=pltpu.CompilerParams(use_tc_tiling_on_sc=False),
)
def _sc_degree(d_hbm, zeros16_hbm, ones16_hbm, out_hbm, d_v, ones_v, acc):
    c = lax.axis_index("c")
    t = lax.axis_index("s")
    w = c * NS + t
    pltpu.sync_copy(d_hbm.at[pl.ds(w * DCH, DCH)], d_v)
    pltpu.sync_copy(ones16_hbm, ones_v)
    pltpu.sync_copy(zeros16_hbm, acc.at[pl.ds(t * RPT, RPT)])
    plsc.subcore_barrier()

    def chunk(i, carry):
        pltpu.sync_copy(ones_v, acc.at[d_v.at[i]], add=True)
        return carry

    lax.fori_loop(0, DCH, chunk, 0)
    plsc.subcore_barrier()
    pltpu.sync_copy(acc.at[pl.ds(t * RPT, RPT)],
                    out_hbm.at[pl.ds(c * NPAD + t * RPT, RPT)])


@functools.partial(
    pl.kernel,
    out_type=jax.ShapeDtypeStruct((NC * NPAD, H), jnp.float32),
    mesh=_mesh,
    scratch_types=[
        pltpu.VMEM((HCH, KS), jnp.int32),
        pltpu.VMEM((HCH, KS), jnp.int32),
        pltpu.VMEM((KS, H), jnp.float32),
        pltpu.VMEM((KS, H), jnp.float32),
        pltpu.VMEM_SHARED((NPAD, H), jnp.float32),
        pltpu.VMEM_SHARED((NPAD, H), jnp.float32),
        pltpu.SemaphoreType.DMA,
        pltpu.SemaphoreType.DMA,
    ],
    compiler_params=pltpu.CompilerParams(use_tc_tiling_on_sc=False),
)
def _sc_scatter(x_hbm, s_hbm, d_hbm, zeros_hbm, out_hbm,
                s_v, d_v, rows0, rows1, xs, acc, sem0, sem1):
    c = lax.axis_index("c")
    t = lax.axis_index("s")
    pltpu.sync_copy(x_hbm.at[pl.ds(c * NPAD + t * RPT, RPT)],
                    xs.at[pl.ds(t * RPT, RPT)])
    pltpu.sync_copy(zeros_hbm, acc.at[pl.ds(t * RPT, RPT)])
    plsc.subcore_barrier()

    for sec in range(SCH // HCH):
        base = t * SCH + sec * HCH
        pltpu.sync_copy(s_hbm.at[pl.ds(base, HCH)], s_v)
        pltpu.sync_copy(d_hbm.at[pl.ds(base, HCH)], d_v)

        pltpu.async_copy(xs.at[s_v.at[0]], rows0, sem0)

        def body(i, carry):
            a = 2 * i
            pltpu.async_copy(xs.at[s_v.at[a + 1]], rows1, sem1)
            pltpu.make_async_copy(xs.at[s_v.at[a]], rows0, sem0).wait()
            pltpu.sync_copy(rows0, acc.at[d_v.at[a]], add=True)

            @pl.when(a + 2 < HCH)
            def _():
                pltpu.async_copy(xs.at[s_v.at[a + 2]], rows0, sem0)

            pltpu.make_async_copy(xs.at[s_v.at[a + 1]], rows1, sem1).wait()
            pltpu.sync_copy(rows1, acc.at[d_v.at[a + 1]], add=True)
            return carry

        lax.fori_loop(0, HCH // 2, body, 0)

    plsc.subcore_barrier()
    pltpu.sync_copy(acc.at[pl.ds(t * RPT, RPT)],
                    out_hbm.at[pl.ds(c * NPAD + t * RPT, RPT)])


def _dinv_from(degp_ref):
    indeg = degp_ref[0:N, 0:1] + degp_ref[NPAD:NPAD + N, 0:1]
    deg = indeg + 1.0
    return lax.rsqrt(jnp.maximum(deg, 1.0))


def _bn_in(h, gamma, beta, eps=1e-3):
    mu = jnp.mean(h, axis=0, keepdims=True)
    var = jnp.mean((h - mu) * (h - mu), axis=0, keepdims=True)
    return (h - mu) * lax.rsqrt(var + eps) * gamma + beta


def _leaky_in(h, alpha=0.3):
    return jnp.where(h > 0, h, alpha * h)


def _join_cols(ref):
    return jnp.concatenate([ref[0:N, :], ref[NPAD:NPAD + N, :]], axis=1)


def _store_split(ref, m):
    ref[0:N, :] = m[:, 0:H]
    ref[NPAD:NPAD + N, :] = m[:, H:D]


def _tc1(x_ref, degp_ref, g1_ref, b1_ref, x1_ref):
    h0 = _bn_in(x_ref[...], g1_ref[...], b1_ref[...])
    _store_split(x1_ref, _dinv_from(degp_ref) * h0)


def _tc2(s1_ref, x1_ref, degp_ref, w1_ref, b1_ref, w2_ref,
         g2_ref, bt2_ref, ws_ref, bs_ref, x2_ref, skip_ref):
    dinv = _dinv_from(degp_ref)
    g0 = dinv * (_join_cols(s1_ref) + _join_cols(x1_ref))
    h1 = _leaky_in(jnp.dot(g0, w1_ref[...],
                           preferred_element_type=jnp.float32) + b1_ref[...])
    h1b = _bn_in(h1, g2_ref[...], bt2_ref[...])
    _store_split(x2_ref, dinv * jnp.dot(h1b, w2_ref[...],
                                        preferred_element_type=jnp.float32))
    skip_ref[...] = jnp.dot(g0, ws_ref[...],
                            preferred_element_type=jnp.float32) + bs_ref[...]


def _tc3(s2_ref, x2_ref, degp_ref, b2_ref, skip_ref, out_ref):
    h = _leaky_in(_dinv_from(degp_ref)
                  * (_join_cols(s2_ref) + _join_cols(x2_ref)) + b2_ref[...])
    out_ref[...] = h + skip_ref[...]


def _tc_call(f, out_shapes):
    return pl.pallas_call(
        f, out_shape=[jax.ShapeDtypeStruct(s, jnp.float32) for s in out_shapes])


@jax.jit
def kernel(x, edge_index, W1, b1, W2, b2, Ws, bs, gamma1, beta1, gamma2,
           beta2):
    ei = edge_index.astype(jnp.int32)
    src, dst = ei[0], ei[1]

    pad_deg = NW * DCH * KD - E
    d_deg = jnp.concatenate([dst, jnp.full((pad_deg,), N, jnp.int32)])
    d_deg = d_deg.reshape(NW * DCH, KD)

    pad_sc = NS * SCH * KS - E
    s_hbm = jnp.concatenate([src, jnp.zeros((pad_sc,), jnp.int32)])
    d_hbm = jnp.concatenate([dst, jnp.full((pad_sc,), N, jnp.int32)])
    s_hbm = s_hbm.reshape(NS * SCH, KS)
    d_hbm = d_hbm.reshape(NS * SCH, KS)

    zeros_rows = jnp.zeros((RPT, H), jnp.float32)
    zeros16 = jnp.zeros((RPT, 16), jnp.float32)
    ones16 = jnp.ones((KD, 16), jnp.float32)

    degp = _sc_degree(d_deg, zeros16, ones16)

    row = lambda v: v.reshape(1, D)
    (x1,) = _tc_call(_tc1, [(NC * NPAD, H)])(x, degp, row(gamma1), row(beta1))
    s1 = _sc_scatter(x1, s_hbm, d_hbm, zeros_rows)
    x2, skip = _tc_call(_tc2, [(NC * NPAD, H), (N, D)])(
        s1, x1, degp, W1, row(b1), W2, row(gamma2), row(beta2), Ws, row(bs))
    s2 = _sc_scatter(x2, s_hbm, d_hbm, zeros_rows)
    (out,) = _tc_call(_tc3, [(N, D)])(s2, x2, degp, row(b2), skip)
    return out
